# Initial kernel scaffold; baseline (speedup 1.0000x reference)
import jax
import jax.numpy as jnp
from jax.experimental import pallas as pl


def kernel(x, weight):
    raise NotImplementedError("write your pallas kernel here")


# speedup vs baseline: 1.5600x; 1.5600x over previous
import functools

import jax
import jax.numpy as jnp
from jax import lax
from jax.experimental import pallas as pl
from jax.experimental.pallas import tpu as pltpu
from jax.experimental.pallas import tpu_sc as plsc

_VOCAB = 1000000
_EMB = 32
_BATCH = 16384
_FIELDS = 26
_N = _BATCH * _FIELDS
_NW = 32
_B_PER_W = _N // _NW
_CHUNK = 1024
_NCHUNKS = _B_PER_W // _CHUNK

_mesh = plsc.VectorSubcoreMesh(core_axis_name="c", subcore_axis_name="s")


@functools.partial(
    pl.kernel,
    mesh=_mesh,
    out_type=jax.ShapeDtypeStruct((_N, _EMB), jnp.float32),
    scratch_types=[
        pltpu.VMEM((_B_PER_W,), jnp.int32),
        pltpu.VMEM((_CHUNK, _EMB), jnp.float32),
        pltpu.SemaphoreType.DMA,
    ],
    compiler_params=pltpu.CompilerParams(use_tc_tiling_on_sc=False),
)
def _emb_lookup(idx_hbm, table_hbm, out_hbm, idx_v, rows_v, gsem):
    wid = lax.axis_index("s") * 2 + lax.axis_index("c")
    base = wid * _B_PER_W
    pltpu.sync_copy(idx_hbm.at[pl.ds(base, _B_PER_W)], idx_v)

    def body(i, carry):
        off = i * _CHUNK
        pltpu.async_copy(
            table_hbm.at[idx_v.at[pl.ds(off, _CHUNK)]], rows_v, gsem
        ).wait()
        pltpu.sync_copy(rows_v, out_hbm.at[pl.ds(base + off, _CHUNK)])
        return carry

    lax.fori_loop(0, _NCHUNKS, body, 0)


def kernel(x, weight):
    out = _emb_lookup(x.reshape(_N), weight)
    return out.reshape(_BATCH, _FIELDS, _EMB)


# speedup vs baseline: 1.5737x; 1.0087x over previous
import functools

import jax
import jax.numpy as jnp
from jax import lax
from jax.experimental import pallas as pl
from jax.experimental.pallas import tpu as pltpu
from jax.experimental.pallas import tpu_sc as plsc

_VOCAB = 1000000
_EMB = 32
_BATCH = 16384
_FIELDS = 26
_N = _BATCH * _FIELDS
_NW = 32
_B_PER_W = _N // _NW
_CHUNK = 1024
_NCHUNKS = _B_PER_W // _CHUNK
_NB = 3

_mesh = plsc.VectorSubcoreMesh(core_axis_name="c", subcore_axis_name="s")


@functools.partial(
    pl.kernel,
    mesh=_mesh,
    out_type=jax.ShapeDtypeStruct((_N, _EMB), jnp.float32),
    scratch_types=[
        pltpu.VMEM((_B_PER_W,), jnp.int32),
        pltpu.VMEM((_NB, _CHUNK, _EMB), jnp.float32),
        pltpu.SemaphoreType.DMA((_NB,)),
        pltpu.SemaphoreType.DMA((_NB,)),
    ],
    compiler_params=pltpu.CompilerParams(use_tc_tiling_on_sc=False),
)
def _emb_lookup(idx_hbm, table_hbm, out_hbm, idx_v, rows_v, gsems, wsems):
    wid = lax.axis_index("s") * 2 + lax.axis_index("c")
    base = wid * _B_PER_W
    pltpu.sync_copy(idx_hbm.at[pl.ds(base, _B_PER_W)], idx_v)

    def gather(j):
        return pltpu.async_copy(
            table_hbm.at[idx_v.at[pl.ds(j * _CHUNK, _CHUNK)]],
            rows_v.at[j % _NB],
            gsems.at[j % _NB],
        )

    def write(j):
        return pltpu.async_copy(
            rows_v.at[j % _NB],
            out_hbm.at[pl.ds(base + j * _CHUNK, _CHUNK)],
            wsems.at[j % _NB],
        )

    gathers, writes = {}, {}
    waited = set()
    for j in range(min(2, _NCHUNKS)):
        gathers[j] = gather(j)
    for i in range(_NCHUNKS):
        gathers[i].wait()
        writes[i] = write(i)
        j = i + 2
        if j < _NCHUNKS:
            if j - _NB >= 0:
                writes[j - _NB].wait()
                waited.add(j - _NB)
            gathers[j] = gather(j)
    for i in range(_NCHUNKS):
        if i not in waited:
            writes[i].wait()


def kernel(x, weight):
    out = _emb_lookup(x.reshape(_N), weight)
    return out.reshape(_BATCH, _FIELDS, _EMB)
